# baseline (speedup 1.0000x reference)
import functools

import jax
import jax.numpy as jnp
from jax import lax
from jax.experimental import pallas as pl
from jax.experimental.pallas import tpu as pltpu
from jax.experimental.pallas import tpu_sc as plsc

D_MODEL = 512
N_ROWS = 119
_NC, _NS = 2, 16
_NW = _NC * _NS
_CHUNK = 128


def _proj_body(cbfv_ref, w_ref, b_ref, out_ref):
    out_ref[...] = (
        jnp.dot(cbfv_ref[...], w_ref[...], preferred_element_type=jnp.float32)
        + b_ref[...]
    )


def _compute_proj(cbfv, W, b):
    return pl.pallas_call(
        _proj_body,
        out_shape=jax.ShapeDtypeStruct((N_ROWS, D_MODEL), jnp.float32),
    )(cbfv, W, b.reshape(1, D_MODEL))


@functools.cache
def _make_gather(B):
    bw = B // _NW
    nch = bw // _CHUNK
    mesh = plsc.VectorSubcoreMesh(core_axis_name="c", subcore_axis_name="s")

    @functools.partial(
        pl.kernel,
        out_type=jax.ShapeDtypeStruct((B, D_MODEL), jnp.float32),
        mesh=mesh,
        scratch_types=[
            pltpu.VMEM((bw,), jnp.int32),
            pltpu.VMEM((_CHUNK, D_MODEL), jnp.float32),
            pltpu.SemaphoreType.DMA,
        ],
    )
    def k(table_hbm, idx_hbm, out_hbm, idx_v, rows_v, sem):
        wid = lax.axis_index("s") * _NC + lax.axis_index("c")
        base = wid * bw
        pltpu.sync_copy(idx_hbm.at[pl.ds(base, bw)], idx_v)

        def body(i, carry):
            off = i * _CHUNK
            pltpu.async_copy(
                table_hbm.at[idx_v.at[pl.ds(off, _CHUNK)]], rows_v, sem
            ).wait()
            pltpu.sync_copy(rows_v, out_hbm.at[pl.ds(base + off, _CHUNK)])
            return carry

        lax.fori_loop(0, nch, body, 0)

    return k


def kernel(src, cbfv, W, b):
    proj = _compute_proj(cbfv, W, b)
    batch, seq = src.shape
    out = _make_gather(batch * seq)(proj, src.reshape(-1))
    return out.reshape(batch, seq, D_MODEL)


# speedup vs baseline: 1.0021x; 1.0021x over previous
import functools

import jax
import jax.numpy as jnp
from jax import lax
from jax.experimental import pallas as pl
from jax.experimental.pallas import tpu as pltpu
from jax.experimental.pallas import tpu_sc as plsc

D_MODEL = 512
N_ROWS = 119
_NC, _NS = 2, 16
_NW = _NC * _NS
_CHUNK = 40
_NBUF = 4


def _proj_body(cbfv_ref, w_ref, b_ref, out_ref):
    out_ref[...] = (
        jnp.dot(cbfv_ref[...], w_ref[...], preferred_element_type=jnp.float32)
        + b_ref[...]
    )


def _compute_proj(cbfv, W, b):
    return pl.pallas_call(
        _proj_body,
        out_shape=jax.ShapeDtypeStruct((N_ROWS, D_MODEL), jnp.float32),
    )(cbfv, W, b.reshape(1, D_MODEL))


@functools.cache
def _make_gather(B):
    bw = B // _NW
    nch = bw // _CHUNK
    rounds = nch // _NBUF
    assert nch % _NBUF == 0 and rounds >= 3
    mesh = plsc.VectorSubcoreMesh(core_axis_name="c", subcore_axis_name="s")

    @functools.partial(
        pl.kernel,
        out_type=jax.ShapeDtypeStruct((B, D_MODEL), jnp.float32),
        mesh=mesh,
        scratch_types=[
            pltpu.VMEM((bw,), jnp.int32),
            [pltpu.VMEM((_CHUNK, D_MODEL), jnp.float32) for _ in range(_NBUF)],
            [pltpu.SemaphoreType.DMA for _ in range(2 * _NBUF)],
        ],
    )
    def k(table_hbm, idx_hbm, out_hbm, idx_v, bufs, sems):
        sins, souts = sems[:_NBUF], sems[_NBUF:]
        wid = lax.axis_index("s") * _NC + lax.axis_index("c")
        base = wid * bw
        pltpu.sync_copy(idx_hbm.at[pl.ds(base, bw)], idx_v)

        def gat(chunk, b):
            return pltpu.make_async_copy(
                table_hbm.at[idx_v.at[pl.ds(chunk * _CHUNK, _CHUNK)]],
                bufs[b], sins[b])

        def outc(chunk, b):
            return pltpu.make_async_copy(
                bufs[b], out_hbm.at[pl.ds(base + chunk * _CHUNK, _CHUNK)],
                souts[b])

        gat(0, 0).start()
        gat(1, 1).start()

        def step(i, b, issue_next, wait_reuse):
            gat(i, b).wait()
            outc(i, b).start()
            if issue_next:
                nb = (b + 2) % _NBUF
                if wait_reuse:
                    outc(i - 2, nb).wait()
                gat(i + 2, nb).start()

        for b in range(_NBUF):
            step(b, b, issue_next=True, wait_reuse=b >= 2)

        def body(j, carry):
            for b in range(_NBUF):
                step(j * _NBUF + b, b, issue_next=True, wait_reuse=True)
            return carry

        lax.fori_loop(1, rounds - 1, body, 0)

        for b in range(_NBUF):
            step((rounds - 1) * _NBUF + b, b, issue_next=b < 2, wait_reuse=True)

        for b in range(_NBUF):
            outc((rounds - 1) * _NBUF + b, b).wait()

    return k


def kernel(src, cbfv, W, b):
    proj = _compute_proj(cbfv, W, b)
    batch, seq = src.shape
    out = _make_gather(batch * seq)(proj, src.reshape(-1))
    return out.reshape(batch, seq, D_MODEL)


# speedup vs baseline: 1.8441x; 1.8403x over previous
import functools

import jax
import jax.numpy as jnp
from jax import lax
from jax.experimental import pallas as pl
from jax.experimental.pallas import tpu as pltpu
from jax.experimental.pallas import tpu_sc as plsc

D_MODEL = 512
N_ROWS = 119
_NC, _NS = 2, 16
_NW = _NC * _NS
_CHUNK = 40
_NBUF = 4


def _proj_body(cbfv_ref, w_ref, b_ref, out_ref):
    out_ref[...] = (
        jnp.dot(cbfv_ref[...], w_ref[...], preferred_element_type=jnp.float32)
        + b_ref[...]
    )


def _compute_proj(cbfv, W, b):
    return pl.pallas_call(
        _proj_body,
        out_shape=jax.ShapeDtypeStruct((N_ROWS, D_MODEL), jnp.float32),
    )(cbfv, W, b.reshape(1, D_MODEL))


@functools.cache
def _make_gather(B):
    bw = B // _NW
    nch = bw // _CHUNK
    assert nch % 2 == 0 and nch >= 6
    mesh = plsc.VectorSubcoreMesh(core_axis_name="c", subcore_axis_name="s")

    @functools.partial(
        pl.kernel,
        out_type=jax.ShapeDtypeStruct((B, D_MODEL), jnp.float32),
        mesh=mesh,
        scratch_types=[
            pltpu.VMEM_SHARED((_NS * bw,), jnp.int32),
            pltpu.VMEM((N_ROWS, D_MODEL), jnp.float32),
            [pltpu.VMEM((_CHUNK, D_MODEL), jnp.float32) for _ in range(2)],
            pltpu.SMEM((2, _CHUNK), jnp.int32),
            [pltpu.SemaphoreType.DMA for _ in range(4)],
        ],
    )
    def k(table_hbm, idx_hbm, out_hbm, idx_sh, table_v, bufs, idx_s, sems):
        souts, sidx = sems[:2], sems[2:]
        sid = lax.axis_index("s")
        wid = sid * _NC + lax.axis_index("c")
        base = wid * bw
        pltpu.sync_copy(table_hbm, table_v)
        pltpu.sync_copy(idx_hbm.at[pl.ds(base, bw)],
                        idx_sh.at[pl.ds(sid * bw, bw)])

        def stage(chunk, b):
            return pltpu.make_async_copy(
                idx_sh.at[pl.ds(sid * bw + chunk * _CHUNK, _CHUNK)],
                idx_s.at[b], sidx[b])

        def outc(chunk, b):
            return pltpu.make_async_copy(
                bufs[b], out_hbm.at[pl.ds(base + chunk * _CHUNK, _CHUNK)],
                souts[b])

        def fill(b):
            def row(r, carry):
                v = idx_s[b, r]
                for c in range(D_MODEL // 16):
                    bufs[b][r, pl.ds(c * 16, 16)] = table_v[v, pl.ds(c * 16, 16)]
                return carry

            lax.fori_loop(0, _CHUNK, row, 0)

        def step(i, b, first, last):
            stage(i, b).wait()
            if not first:
                outc(i - 2, b).wait()
            fill(b)
            outc(i, b).start()
            if not last:
                stage(i + 2, b).start()

        stage(0, 0).start()
        stage(1, 1).start()
        step(0, 0, first=True, last=False)
        step(1, 1, first=True, last=False)

        def body(j, carry):
            for b in range(2):
                step(2 * j + b, b, first=False, last=False)
            return carry

        lax.fori_loop(1, nch // 2 - 1, body, 0)

        step(nch - 2, 0, first=False, last=True)
        step(nch - 1, 1, first=False, last=True)
        outc(nch - 2, 0).wait()
        outc(nch - 1, 1).wait()

    return k


def kernel(src, cbfv, W, b):
    proj = _compute_proj(cbfv, W, b)
    batch, seq = src.shape
    out = _make_gather(batch * seq)(proj, src.T.reshape(-1))
    return out.reshape(seq, batch, D_MODEL).transpose(1, 0, 2)


# speedup vs baseline: 5.0705x; 2.7495x over previous
import functools

import jax
import jax.numpy as jnp
from jax import lax
from jax.experimental import pallas as pl
from jax.experimental.pallas import tpu as pltpu
from jax.experimental.pallas import tpu_sc as plsc

D_MODEL = 512
N_ROWS = 119
_NC, _NS = 2, 16
_NW = _NC * _NS
_CHUNK = 40
_NBUF = 4


def _proj_body(cbfv_ref, w_ref, b_ref, out_ref):
    out_ref[...] = (
        jnp.dot(cbfv_ref[...], w_ref[...], preferred_element_type=jnp.float32)
        + b_ref[...]
    )


def _compute_proj(cbfv, W, b):
    return pl.pallas_call(
        _proj_body,
        out_shape=jax.ShapeDtypeStruct((N_ROWS, D_MODEL), jnp.float32),
    )(cbfv, W, b.reshape(1, D_MODEL))


@functools.cache
def _make_gather(B):
    bw = B // _NW
    nch = bw // _CHUNK
    assert nch % 2 == 0 and nch >= 6
    mesh = plsc.VectorSubcoreMesh(core_axis_name="c", subcore_axis_name="s")

    @functools.partial(
        pl.kernel,
        out_type=jax.ShapeDtypeStruct((B, D_MODEL), jnp.float32),
        mesh=mesh,
        scratch_types=[
            pltpu.VMEM_SHARED((_NS * bw,), jnp.int32),
            pltpu.VMEM((N_ROWS, D_MODEL), jnp.float32),
            [pltpu.VMEM((_CHUNK, D_MODEL), jnp.float32) for _ in range(2)],
            pltpu.SMEM((2, _CHUNK), jnp.int32),
            [pltpu.SemaphoreType.DMA for _ in range(4)],
        ],
    )
    def k(table_hbm, idx_hbm, out_hbm, idx_sh, table_v, bufs, idx_s, sems):
        souts, sidx = sems[:2], sems[2:]
        sid = lax.axis_index("s")
        wid = sid * _NC + lax.axis_index("c")
        base = wid * bw
        pltpu.sync_copy(table_hbm, table_v)
        pltpu.sync_copy(idx_hbm.at[pl.ds(base, bw)],
                        idx_sh.at[pl.ds(sid * bw, bw)])

        def stage(chunk, b):
            return pltpu.make_async_copy(
                idx_sh.at[pl.ds(sid * bw + chunk * _CHUNK, _CHUNK)],
                idx_s.at[b], sidx[b])

        def outc(chunk, b):
            return pltpu.make_async_copy(
                bufs[b], out_hbm.at[pl.ds(base + chunk * _CHUNK, _CHUNK)],
                souts[b])

        def fill(b):
            def row(r, carry):
                v = idx_s[b, r]
                vals = [table_v[v, pl.ds(c * 16, 16)]
                        for c in range(D_MODEL // 16)]
                for c in range(D_MODEL // 16):
                    bufs[b][r, pl.ds(c * 16, 16)] = vals[c]
                return carry

            lax.fori_loop(0, _CHUNK, row, 0, unroll=2)

        def step(i, b, first, last):
            stage(i, b).wait()
            if not first:
                outc(i - 2, b).wait()
            fill(b)
            outc(i, b).start()
            if not last:
                stage(i + 2, b).start()

        stage(0, 0).start()
        stage(1, 1).start()
        step(0, 0, first=True, last=False)
        step(1, 1, first=True, last=False)

        def body(j, carry):
            for b in range(2):
                step(2 * j + b, b, first=False, last=False)
            return carry

        lax.fori_loop(1, nch // 2 - 1, body, 0)

        step(nch - 2, 0, first=False, last=True)
        step(nch - 1, 1, first=False, last=True)
        outc(nch - 2, 0).wait()
        outc(nch - 1, 1).wait()

    return k


def kernel(src, cbfv, W, b):
    proj = _compute_proj(cbfv, W, b)
    batch, seq = src.shape
    out = _make_gather(batch * seq)(proj, src.T.reshape(-1))
    return out.reshape(seq, batch, D_MODEL).transpose(1, 0, 2)
